# Initial kernel scaffold; baseline (speedup 1.0000x reference)
import jax
import jax.numpy as jnp
from jax.experimental import pallas as pl


def kernel(pred_scores, pred_rboxes, anchor_points, gt_labels, gt_rboxes, gt_crowd, pad_gt_mask, bg_index):
    raise NotImplementedError("write your pallas kernel here")


# speedup vs baseline: 13.8112x; 13.8112x over previous
import jax
import jax.numpy as jnp
from jax import lax
from jax.experimental import pallas as pl
from jax.experimental.pallas import tpu as pltpu

_TOPK = 13
_EPS = 1e-09
_IOU_EPS = 1e-07


def _covariance(w, h, r):
    a = w * w / 12.0
    b = h * h / 12.0
    cos = jnp.cos(r)
    sin = jnp.sin(r)
    cos2 = cos * cos
    sin2 = sin * sin
    return a * cos2 + b * sin2, a * sin2 + b * cos2, (a - b) * cos * sin


def _assigner_kernel(ps_ref, prT_ref, gtl_ref, gtr_ref, gtc_ref, bg_ref,
                     scores_ref, packed_ref):
    n = gtr_ref.shape[1]
    L = ps_ref.shape[1]
    C = ps_ref.shape[2]
    f32 = jnp.float32

    gtr = gtr_ref[0]
    gx = gtr[:, 0:1]
    gy = gtr[:, 1:2]
    a1, b1, c1 = _covariance(gtr[:, 2:3], gtr[:, 3:4], gtr[:, 4:5])

    px = prT_ref[0, 0:1, :]
    py = prT_ref[0, 1:2, :]
    a2, b2, c2 = _covariance(prT_ref[0, 2:3, :], prT_ref[0, 3:4, :],
                             prT_ref[0, 4:5, :])

    sa = a1 + a2
    sb = b1 + b2
    sc_ = c1 + c2
    den = sa * sb - sc_ * sc_ + _IOU_EPS
    dy = gy - py
    dx = gx - px
    t1 = (sa * (dy * dy) + sb * (dx * dx)) / den * 0.25
    t2 = sc_ * (px - gx) * (gy - py) / den * 0.5
    num3 = sa * sb - sc_ * sc_
    det1 = jnp.maximum(a1 * b1 - c1 * c1, 0.0)
    det2 = jnp.maximum(a2 * b2 - c2 * c2, 0.0)
    t3 = jnp.log(num3 / (4.0 * jnp.sqrt(det1 * det2) + _IOU_EPS) + _IOU_EPS) * 0.5
    bd = jnp.clip(t1 + t2 + t3, _IOU_EPS, 100.0)
    hd = jnp.sqrt(1.0 - jnp.exp(-bd) + _IOU_EPS)
    iou = 1.0 - hd

    gtl = gtl_ref[0]
    cio = lax.broadcasted_iota(jnp.int32, (1, C), 1).astype(f32)
    onehot = (gtl == cio).astype(f32)
    cls = lax.dot_general(onehot, ps_ref[0], (((1,), (1,)), ((), ())),
                          precision=lax.Precision.HIGHEST,
                          preferred_element_type=f32)

    iou2 = iou * iou
    iou6 = iou2 * iou2 * iou2
    align = cls * iou6

    lio = lax.broadcasted_iota(jnp.int32, (n, L), 1)

    def topk_body(_, carry):
        metric, mask = carry
        m = jnp.max(metric, axis=1, keepdims=True)
        is_m = metric == m
        first = jnp.min(jnp.where(is_m, lio, L), axis=1, keepdims=True)
        sel = lio == first
        mask = jnp.where(sel, 1.0, mask)
        metric = jnp.where(sel, -1.0, metric)
        return metric, mask

    _, mask = lax.fori_loop(0, _TOPK, topk_body,
                            (align, jnp.zeros((n, L), f32)))

    colsum = jnp.sum(mask, axis=0, keepdims=True)
    gio = lax.broadcasted_iota(jnp.int32, (n, L), 0)
    miou = jnp.max(iou, axis=0, keepdims=True)
    first_g = jnp.min(jnp.where(iou == miou, gio, n), axis=0, keepdims=True)
    is_max_iou = (gio == first_g).astype(f32)
    mask_pos = jnp.where(colsum > 1.0, is_max_iou, mask)

    am = align * mask_pos
    m_am = jnp.max(am, axis=1, keepdims=True)
    m_iou = jnp.max(iou * mask_pos, axis=1, keepdims=True)
    am_scaled = am / (m_am + _EPS) * m_iou

    gio_col = lax.broadcasted_iota(jnp.int32, (n, 1), 0).astype(f32)
    gtc = gtc_ref[0]
    ones_col = jnp.ones((n, 1), f32)
    table = jnp.concatenate(
        [gtr, gtl, gtc, gio_col, ones_col], axis=1)
    out_t = lax.dot_general(mask_pos, table, (((0,), (0,)), ((), ())),
                            precision=lax.Precision.HIGHEST,
                            preferred_element_type=f32)
    mult = lax.dot_general(am_scaled, ones_col, (((0,), (0,)), ((), ())),
                           precision=lax.Precision.HIGHEST,
                           preferred_element_type=f32)

    rbox_g = out_t[:, 0:5]
    lab_g = out_t[:, 5:6]
    crowd_g = out_t[:, 6:7]
    idx_g = out_t[:, 7:8]
    pos = out_t[:, 8:9] > 0.5

    bg_f = bg_ref[0, 0].astype(f32)
    lab_f = jnp.where(pos, lab_g, bg_f)
    rbox_o = jnp.where(pos, rbox_g, gtr[0:1, :])
    crowd_f = jnp.where(pos, crowd_g, gtc[0:1, :])

    kio = lax.broadcasted_iota(jnp.int32, (1, C), 1).astype(f32)
    keep = kio + (kio >= bg_f).astype(f32)
    sc = (lab_f == keep).astype(f32) * mult
    sc = jnp.where(crowd_f > 0.5, 0.0, sc)

    scores_ref[0] = sc
    packed_ref[0] = jnp.concatenate(
        [rbox_o, lab_f, idx_g, crowd_f], axis=1)


def kernel(pred_scores, pred_rboxes, anchor_points, gt_labels, gt_rboxes,
           gt_crowd, pad_gt_mask, bg_index):
    B, L, C = pred_scores.shape
    n = gt_rboxes.shape[1]
    prT = jnp.transpose(pred_rboxes, (0, 2, 1))
    gtl_f = gt_labels.astype(jnp.float32)
    gtc_f = gt_crowd.astype(jnp.float32)
    bg = jnp.asarray(bg_index, jnp.int32).reshape(1, 1)

    scores, packed = pl.pallas_call(
        _assigner_kernel,
        grid=(B,),
        in_specs=[
            pl.BlockSpec((1, L, C), lambda b: (b, 0, 0)),
            pl.BlockSpec((1, 5, L), lambda b: (b, 0, 0)),
            pl.BlockSpec((1, n, 1), lambda b: (b, 0, 0)),
            pl.BlockSpec((1, n, 5), lambda b: (b, 0, 0)),
            pl.BlockSpec((1, n, 1), lambda b: (b, 0, 0)),
            pl.BlockSpec(memory_space=pltpu.SMEM),
        ],
        out_specs=[
            pl.BlockSpec((1, L, C), lambda b: (b, 0, 0)),
            pl.BlockSpec((1, L, 8), lambda b: (b, 0, 0)),
        ],
        out_shape=[
            jax.ShapeDtypeStruct((B, L, C), jnp.float32),
            jax.ShapeDtypeStruct((B, L, 8), jnp.float32),
        ],
    )(pred_scores, prT, gtl_f, gt_rboxes, gtc_f, bg)

    assigned_rboxes = packed[:, :, 0:5]
    assigned_labels = packed[:, :, 5].astype(jnp.int32)
    assigned_gt_index = packed[:, :, 6].astype(jnp.int32)
    assigned_crowd = packed[:, :, 7] > 0.5
    return (assigned_labels, assigned_rboxes, scores, assigned_gt_index,
            assigned_crowd)


# speedup vs baseline: 30.6459x; 2.2189x over previous
import jax
import jax.numpy as jnp
from jax import lax
from jax.experimental import pallas as pl
from jax.experimental.pallas import tpu as pltpu

_TOPK = 13
_EPS = 1e-09
_IOU_EPS = 1e-07


def _covariance(w, h, r):
    a = w * w / 12.0
    b = h * h / 12.0
    cos = jnp.cos(r)
    sin = jnp.sin(r)
    cos2 = cos * cos
    sin2 = sin * sin
    return a * cos2 + b * sin2, a * sin2 + b * cos2, (a - b) * cos * sin


def _assigner_kernel(psT_ref, prT_ref, gtl_ref, gtr_ref, gtrT_ref,
                     gtlrow_ref, gtcrow_ref, bg_ref, scores_ref, packed_ref):
    n = gtr_ref.shape[1]
    L = psT_ref.shape[2]
    C = psT_ref.shape[1]
    f32 = jnp.float32

    gtr = gtr_ref[0]
    gx = gtr[:, 0:1]
    gy = gtr[:, 1:2]
    a1, b1, c1 = _covariance(gtr[:, 2:3], gtr[:, 3:4], gtr[:, 4:5])

    px = prT_ref[0, 0:1, :]
    py = prT_ref[0, 1:2, :]
    a2, b2, c2 = _covariance(prT_ref[0, 2:3, :], prT_ref[0, 3:4, :],
                             prT_ref[0, 4:5, :])

    sa = a1 + a2
    sb = b1 + b2
    sc_ = c1 + c2
    den = sa * sb - sc_ * sc_ + _IOU_EPS
    dy = gy - py
    dx = gx - px
    t1 = (sa * (dy * dy) + sb * (dx * dx)) / den * 0.25
    t2 = sc_ * (px - gx) * (gy - py) / den * 0.5
    num3 = sa * sb - sc_ * sc_
    det1 = jnp.maximum(a1 * b1 - c1 * c1, 0.0)
    det2 = jnp.maximum(a2 * b2 - c2 * c2, 0.0)
    t3 = jnp.log(num3 / (4.0 * jnp.sqrt(det1 * det2) + _IOU_EPS) + _IOU_EPS) * 0.5
    bd = jnp.clip(t1 + t2 + t3, _IOU_EPS, 100.0)
    hd = jnp.sqrt(1.0 - jnp.exp(-bd) + _IOU_EPS)
    iou = 1.0 - hd

    gtl = gtl_ref[0]
    cio = lax.broadcasted_iota(jnp.int32, (1, C), 1).astype(f32)
    onehot = (gtl == cio).astype(f32)
    cls = lax.dot_general(onehot, psT_ref[0], (((1,), (0,)), ((), ())),
                          precision=lax.Precision.HIGHEST,
                          preferred_element_type=f32)

    iou2 = iou * iou
    iou6 = iou2 * iou2 * iou2
    align = cls * iou6

    lio = lax.broadcasted_iota(jnp.int32, (n, L), 1)

    def topk_body(_, metric):
        m = jnp.max(metric, axis=1, keepdims=True)
        first = jnp.min(jnp.where(metric == m, lio, L), axis=1, keepdims=True)
        return jnp.where(lio == first, -1.0, metric)

    metric_f = lax.fori_loop(0, _TOPK, topk_body, align)
    mask = (metric_f == -1.0).astype(f32)

    colsum = jnp.sum(mask, axis=0, keepdims=True)
    gio = lax.broadcasted_iota(jnp.int32, (n, L), 0)
    miou = jnp.max(iou, axis=0, keepdims=True)
    first_g = jnp.min(jnp.where(iou == miou, gio, n), axis=0, keepdims=True)
    is_max_iou = (gio == first_g).astype(f32)
    mask_pos = jnp.where(colsum > 1.0, is_max_iou, mask)

    am = align * mask_pos
    m_am = jnp.max(am, axis=1, keepdims=True)
    m_iou = jnp.max(iou * mask_pos, axis=1, keepdims=True)
    am_scaled = am / (m_am + _EPS) * m_iou

    cnt_row = jnp.sum(mask_pos, axis=0, keepdims=True)
    mult_row = jnp.max(am_scaled, axis=0, keepdims=True)

    nio = lax.broadcasted_iota(jnp.int32, (1, n), 1).astype(f32)
    tableT = jnp.concatenate(
        [gtrT_ref[0], gtlrow_ref[0], gtcrow_ref[0], nio], axis=0)
    out8 = lax.dot_general(tableT, mask_pos, (((1,), (0,)), ((), ())),
                           precision=lax.Precision.HIGHEST,
                           preferred_element_type=f32)

    pos_row = cnt_row > 0.5
    bg_f = bg_ref[0, 0].astype(f32)
    lab_row = jnp.where(pos_row, out8[5:6, :], bg_f)
    rboxT = jnp.where(pos_row, out8[0:5, :], gtrT_ref[0][:, 0:1])
    crowd_row = jnp.where(pos_row, out8[6:7, :], gtcrow_ref[0][:, 0:1])
    idx_row = out8[7:8, :]

    kio = lax.broadcasted_iota(jnp.int32, (C, 1), 0).astype(f32)
    keep = kio + (kio >= bg_f).astype(f32)
    sc = (keep == lab_row).astype(f32) * mult_row
    sc = jnp.where(crowd_row > 0.5, 0.0, sc)

    scores_ref[0] = sc
    packed_ref[0] = jnp.concatenate(
        [rboxT, lab_row, idx_row, crowd_row], axis=0)


def kernel(pred_scores, pred_rboxes, anchor_points, gt_labels, gt_rboxes,
           gt_crowd, pad_gt_mask, bg_index):
    B, L, C = pred_scores.shape
    n = gt_rboxes.shape[1]
    psT = jnp.transpose(pred_scores, (0, 2, 1))
    prT = jnp.transpose(pred_rboxes, (0, 2, 1))
    gtrT = jnp.transpose(gt_rboxes, (0, 2, 1))
    gtl_f = gt_labels.astype(jnp.float32)
    gtl_row = jnp.transpose(gtl_f, (0, 2, 1))
    gtc_row = jnp.transpose(gt_crowd.astype(jnp.float32), (0, 2, 1))
    bg = jnp.asarray(bg_index, jnp.int32).reshape(1, 1)

    scoresT, packed = pl.pallas_call(
        _assigner_kernel,
        grid=(B,),
        in_specs=[
            pl.BlockSpec((1, C, L), lambda b: (b, 0, 0)),
            pl.BlockSpec((1, 5, L), lambda b: (b, 0, 0)),
            pl.BlockSpec((1, n, 1), lambda b: (b, 0, 0)),
            pl.BlockSpec((1, n, 5), lambda b: (b, 0, 0)),
            pl.BlockSpec((1, 5, n), lambda b: (b, 0, 0)),
            pl.BlockSpec((1, 1, n), lambda b: (b, 0, 0)),
            pl.BlockSpec((1, 1, n), lambda b: (b, 0, 0)),
            pl.BlockSpec(memory_space=pltpu.SMEM),
        ],
        out_specs=[
            pl.BlockSpec((1, C, L), lambda b: (b, 0, 0)),
            pl.BlockSpec((1, 8, L), lambda b: (b, 0, 0)),
        ],
        out_shape=[
            jax.ShapeDtypeStruct((B, C, L), jnp.float32),
            jax.ShapeDtypeStruct((B, 8, L), jnp.float32),
        ],
    )(psT, prT, gtl_f, gt_rboxes, gtrT, gtl_row, gtc_row, bg)

    assigned_scores = jnp.transpose(scoresT, (0, 2, 1))
    assigned_rboxes = jnp.transpose(packed[:, 0:5, :], (0, 2, 1))
    assigned_labels = packed[:, 5, :].astype(jnp.int32)
    assigned_gt_index = packed[:, 6, :].astype(jnp.int32)
    assigned_crowd = packed[:, 7, :] > 0.5
    return (assigned_labels, assigned_rboxes, assigned_scores,
            assigned_gt_index, assigned_crowd)
